# baseline (speedup 1.0000x reference)
import jax
import jax.numpy as jnp
from jax.experimental import pallas as pl
from jax.experimental.pallas import tpu as pltpu

_CHUNKS = (128, 256, 256, 384, 512, 768, 896, 896)


def _fused_kernel(s_hbm, e_hbm, c_hbm, out_ref,
                  s_v, e_v, c_v, sem_e, sem_s, sem_c):
    cp_c = pltpu.make_async_copy(c_hbm, c_v, sem_c)
    cp_c.start()
    offs = []
    off = 0
    copies_e = []
    for i, ch in enumerate(_CHUNKS):
        cp = pltpu.make_async_copy(
            e_hbm.at[pl.ds(off, ch), :],
            e_v.at[pl.ds(off, ch), :],
            sem_e.at[i],
        )
        cp.start()
        copies_e.append(cp)
        offs.append(off)
        off += ch
    cp_s = pltpu.make_async_copy(s_hbm, s_v, sem_s)
    cp_s.start()

    cp_c.wait()
    t = None
    for i, ch in enumerate(_CHUNKS):
        copies_e[i].wait()
        part = jax.lax.dot_general(
            c_v[:, offs[i]:offs[i] + ch],
            e_v[offs[i]:offs[i] + ch, :],
            (((1,), (0,)), ((), ())),
            preferred_element_type=jnp.float32,
        )
        t = part if t is None else t + part

    cp_s.wait()
    out_ref[...] = jax.lax.dot_general(
        s_v[...], t,
        (((1,), (1,)), ((), ())),
        preferred_element_type=jnp.float32,
    )


def kernel(samples, enc_weight, cent_weight):
    batch, n_features = samples.shape
    n_classes, n_dims = cent_weight.shape
    assert sum(_CHUNKS) == n_dims
    return pl.pallas_call(
        _fused_kernel,
        in_specs=[
            pl.BlockSpec(memory_space=pl.ANY),
            pl.BlockSpec(memory_space=pl.ANY),
            pl.BlockSpec(memory_space=pl.ANY),
        ],
        out_specs=pl.BlockSpec(memory_space=pltpu.VMEM),
        out_shape=jax.ShapeDtypeStruct((batch, n_classes), jnp.float32),
        scratch_shapes=[
            pltpu.VMEM((batch, n_features), jnp.float32),
            pltpu.VMEM((n_dims, n_features), jnp.float32),
            pltpu.VMEM((n_classes, n_dims), jnp.float32),
            pltpu.SemaphoreType.DMA((len(_CHUNKS),)),
            pltpu.SemaphoreType.DMA,
            pltpu.SemaphoreType.DMA,
        ],
    )(samples, enc_weight, cent_weight)


# speedup vs baseline: 1.0111x; 1.0111x over previous
import jax
import jax.numpy as jnp
from jax.experimental import pallas as pl
from jax.experimental.pallas import tpu as pltpu

_CHUNKS = (128, 256, 256, 384, 512, 640, 896, 1024)


def _fused_kernel(s_hbm, e_hbm, c_hbm, out_ref,
                  s_v, e_v, c_v, sem_e, sem_s, sem_c):
    cp_c = pltpu.make_async_copy(c_hbm, c_v, sem_c)
    cp_c.start()
    offs = []
    off = 0
    copies_e = []
    for i, ch in enumerate(_CHUNKS):
        cp = pltpu.make_async_copy(
            e_hbm.at[pl.ds(off, ch), :],
            e_v.at[pl.ds(off, ch), :],
            sem_e.at[i],
        )
        cp.start()
        copies_e.append(cp)
        offs.append(off)
        off += ch
    cp_s = pltpu.make_async_copy(s_hbm, s_v, sem_s)
    cp_s.start()

    cp_c.wait()
    t = None
    for i, ch in enumerate(_CHUNKS):
        copies_e[i].wait()
        part = jax.lax.dot_general(
            c_v[:, offs[i]:offs[i] + ch],
            e_v[offs[i]:offs[i] + ch, :],
            (((1,), (0,)), ((), ())),
            preferred_element_type=jnp.float32,
        )
        t = part if t is None else t + part

    cp_s.wait()
    out_ref[...] = jax.lax.dot_general(
        s_v[...], t,
        (((1,), (1,)), ((), ())),
        preferred_element_type=jnp.float32,
    )


def kernel(samples, enc_weight, cent_weight):
    batch, n_features = samples.shape
    n_classes, n_dims = cent_weight.shape
    assert sum(_CHUNKS) == n_dims
    return pl.pallas_call(
        _fused_kernel,
        in_specs=[
            pl.BlockSpec(memory_space=pl.ANY),
            pl.BlockSpec(memory_space=pl.ANY),
            pl.BlockSpec(memory_space=pl.ANY),
        ],
        out_specs=pl.BlockSpec(memory_space=pltpu.VMEM),
        out_shape=jax.ShapeDtypeStruct((batch, n_classes), jnp.float32),
        scratch_shapes=[
            pltpu.VMEM((batch, n_features), jnp.float32),
            pltpu.VMEM((n_dims, n_features), jnp.float32),
            pltpu.VMEM((n_classes, n_dims), jnp.float32),
            pltpu.SemaphoreType.DMA((len(_CHUNKS),)),
            pltpu.SemaphoreType.DMA,
            pltpu.SemaphoreType.DMA,
        ],
    )(samples, enc_weight, cent_weight)
